# baseline (device time: 134021 ns/iter reference)
import jax
import jax.numpy as jnp
from jax import lax
from jax.experimental import pallas as pl
from jax.experimental.pallas import tpu as pltpu

N_DEV = 32
N_STEPS = 5
B, Sq, Hq, Hkv, Dh = 2, 256, 8, 2, 64
D_MODEL = 768
DQ = Hq * Dh
GROUP = Hq // Hkv
SCALE = 0.125

BF16 = jnp.bfloat16
F32 = jnp.float32


def kernel(x, Wq, Wo, K_ext, V_ext):
    skv = K_ext.shape[1]

    def body(
        x_ref, wq_ref, wo_ref, k_ref, v_ref, out_ref,
        q_buf, acc_o, acc_l, norm_buf, recv_o, recv_l,
        send_sem_o, recv_sem_o, send_sem_l, recv_sem_l,
    ):
        me = lax.axis_index("i")

        barrier = pltpu.get_barrier_semaphore()
        for k in range(N_STEPS):
            pl.semaphore_signal(
                barrier, inc=1,
                device_id=(me ^ (1 << k),),
                device_id_type=pl.DeviceIdType.MESH,
            )
        pl.semaphore_wait(barrier, N_STEPS)

        wq = wq_ref[...].astype(BF16)
        for b in range(B):
            q_buf[b] = lax.dot_general(
                x_ref[b].astype(BF16), wq,
                (((1,), (0,)), ((), ())),
                preferred_element_type=F32,
            ).astype(BF16)

        for b in range(B):
            for h in range(Hq):
                q = q_buf[b, :, h * Dh:(h + 1) * Dh]
                kk = k_ref[b, :, h // GROUP, :].astype(BF16)
                vv = v_ref[b, :, h // GROUP, :].astype(BF16)
                s = lax.dot_general(
                    q, kk, (((1,), (1,)), ((), ())),
                    preferred_element_type=F32,
                ) * SCALE
                p = jnp.exp(s)
                acc_l[b, :, h:h + 1] = jnp.sum(p, axis=1, keepdims=True)
                acc_o[b, :, h * Dh:(h + 1) * Dh] = lax.dot_general(
                    p.astype(BF16), vv, (((1,), (0,)), ((), ())),
                    preferred_element_type=F32,
                )

        for k in range(N_STEPS):
            partner = me ^ (1 << k)
            ro = pltpu.make_async_remote_copy(
                src_ref=acc_o, dst_ref=recv_o.at[k],
                send_sem=send_sem_o.at[k], recv_sem=recv_sem_o.at[k],
                device_id=(partner,), device_id_type=pl.DeviceIdType.MESH,
            )
            rl = pltpu.make_async_remote_copy(
                src_ref=acc_l, dst_ref=recv_l.at[k],
                send_sem=send_sem_l.at[k], recv_sem=recv_sem_l.at[k],
                device_id=(partner,), device_id_type=pl.DeviceIdType.MESH,
            )
            ro.start()
            rl.start()
            ro.wait()
            rl.wait()
            acc_o[...] = acc_o[...] + recv_o[k]
            acc_l[...] = acc_l[...] + recv_l[k]

        wo = wo_ref[...].astype(BF16)
        for b in range(B):
            for h in range(Hq):
                norm_buf[b, :, h * Dh:(h + 1) * Dh] = (
                    acc_o[b, :, h * Dh:(h + 1) * Dh] / acc_l[b, :, h:h + 1]
                ).astype(BF16)
            out_ref[b] = lax.dot_general(
                norm_buf[b], wo, (((1,), (0,)), ((), ())),
                preferred_element_type=F32,
            )

    return pl.pallas_call(
        body,
        out_shape=jax.ShapeDtypeStruct((B, Sq, D_MODEL), F32),
        in_specs=[pl.BlockSpec(memory_space=pltpu.VMEM)] * 5,
        out_specs=pl.BlockSpec(memory_space=pltpu.VMEM),
        scratch_shapes=[
            pltpu.VMEM((B, Sq, DQ), BF16),
            pltpu.VMEM((B, Sq, DQ), F32),
            pltpu.VMEM((B, Sq, Hq), F32),
            pltpu.VMEM((B, Sq, DQ), BF16),
            pltpu.VMEM((N_STEPS, B, Sq, DQ), F32),
            pltpu.VMEM((N_STEPS, B, Sq, Hq), F32),
            pltpu.SemaphoreType.DMA((N_STEPS,)),
            pltpu.SemaphoreType.DMA((N_STEPS,)),
            pltpu.SemaphoreType.DMA((N_STEPS,)),
            pltpu.SemaphoreType.DMA((N_STEPS,)),
        ],
        compiler_params=pltpu.CompilerParams(collective_id=0),
    )(x, Wq, Wo, K_ext, V_ext)


# device time: 93400 ns/iter; 1.4349x vs baseline; 1.4349x over previous
import jax
import jax.numpy as jnp
from jax import lax
from jax.experimental import pallas as pl
from jax.experimental.pallas import tpu as pltpu

N_DEV = 32
N_STEPS = 5
B, Sq, Hq, Hkv, Dh = 2, 256, 8, 2, 64
D_MODEL = 768
DQ = Hq * Dh
GROUP = Hq // Hkv
SCALE = 0.125

BF16 = jnp.bfloat16
F32 = jnp.float32


def kernel(x, Wq, Wo, K_ext, V_ext):
    skv = K_ext.shape[1]

    def body(
        x_ref, wq_ref, wo_ref, k_ref, v_ref, out_ref,
        q_buf, acc_o, acc_l, norm_buf, send_o, recv_o, recv_l,
        send_sem_o, recv_sem_o, send_sem_l, recv_sem_l,
    ):
        me = lax.axis_index("i")

        barrier = pltpu.get_barrier_semaphore()
        for k in range(N_STEPS):
            pl.semaphore_signal(
                barrier, inc=1,
                device_id=(me ^ (1 << k),),
                device_id_type=pl.DeviceIdType.MESH,
            )

        wq = wq_ref[...].astype(BF16)
        for b in range(B):
            q_buf[b] = lax.dot_general(
                x_ref[b].astype(BF16), wq,
                (((1,), (0,)), ((), ())),
                preferred_element_type=F32,
            ).astype(BF16)

        for b in range(B):
            for h in range(Hq):
                q = q_buf[b, :, h * Dh:(h + 1) * Dh]
                kk = k_ref[b, :, h // GROUP, :].astype(BF16)
                vv = v_ref[b, :, h // GROUP, :].astype(BF16)
                s = lax.dot_general(
                    q, kk, (((1,), (1,)), ((), ())),
                    preferred_element_type=F32,
                ) * SCALE
                p = jnp.exp(s)
                acc_l[b, :, h:h + 1] = jnp.sum(p, axis=1, keepdims=True)
                acc_o[b, :, h * Dh:(h + 1) * Dh] = lax.dot_general(
                    p.astype(BF16), vv, (((1,), (0,)), ((), ())),
                    preferred_element_type=F32,
                )

        pl.semaphore_wait(barrier, N_STEPS)

        for k in range(N_STEPS):
            partner = me ^ (1 << k)
            send_o[...] = acc_o[...].astype(BF16)
            ro = pltpu.make_async_remote_copy(
                src_ref=send_o, dst_ref=recv_o.at[k],
                send_sem=send_sem_o.at[k], recv_sem=recv_sem_o.at[k],
                device_id=(partner,), device_id_type=pl.DeviceIdType.MESH,
            )
            rl = pltpu.make_async_remote_copy(
                src_ref=acc_l, dst_ref=recv_l.at[k],
                send_sem=send_sem_l.at[k], recv_sem=recv_sem_l.at[k],
                device_id=(partner,), device_id_type=pl.DeviceIdType.MESH,
            )
            ro.start()
            rl.start()
            ro.wait()
            rl.wait()
            acc_o[...] = acc_o[...] + recv_o[k].astype(F32)
            acc_l[...] = acc_l[...] + recv_l[k]

        wo = wo_ref[...].astype(BF16)
        for b in range(B):
            for h in range(Hq):
                norm_buf[b, :, h * Dh:(h + 1) * Dh] = (
                    acc_o[b, :, h * Dh:(h + 1) * Dh] / acc_l[b, :, h:h + 1]
                ).astype(BF16)
            out_ref[b] = lax.dot_general(
                norm_buf[b], wo, (((1,), (0,)), ((), ())),
                preferred_element_type=F32,
            )

    return pl.pallas_call(
        body,
        out_shape=jax.ShapeDtypeStruct((B, Sq, D_MODEL), F32),
        in_specs=[pl.BlockSpec(memory_space=pltpu.VMEM)] * 5,
        out_specs=pl.BlockSpec(memory_space=pltpu.VMEM),
        scratch_shapes=[
            pltpu.VMEM((B, Sq, DQ), BF16),
            pltpu.VMEM((B, Sq, DQ), F32),
            pltpu.VMEM((B, Sq, Hq), F32),
            pltpu.VMEM((B, Sq, DQ), BF16),
            pltpu.VMEM((B, Sq, DQ), BF16),
            pltpu.VMEM((N_STEPS, B, Sq, DQ), BF16),
            pltpu.VMEM((N_STEPS, B, Sq, Hq), F32),
            pltpu.SemaphoreType.DMA((N_STEPS,)),
            pltpu.SemaphoreType.DMA((N_STEPS,)),
            pltpu.SemaphoreType.DMA((N_STEPS,)),
            pltpu.SemaphoreType.DMA((N_STEPS,)),
        ],
        compiler_params=pltpu.CompilerParams(collective_id=0),
    )(x, Wq, Wo, K_ext, V_ext)


# device time: 15515 ns/iter; 8.6382x vs baseline; 6.0200x over previous
import jax
import jax.numpy as jnp
from jax import lax
from jax.experimental import pallas as pl
from jax.experimental.pallas import tpu as pltpu

N_DEV = 32
N_STEPS = 5
B, Sq, Hq, Hkv, Dh = 2, 256, 8, 2, 64
D_MODEL = 768
DQ = Hq * Dh
GROUP = Hq // Hkv
DE = Dh + 8
DQE = Hq * DE
SCALE = 0.125

BF16 = jnp.bfloat16
F32 = jnp.float32


def kernel(x, Wq, Wo, K_ext, V_ext):
    skv = K_ext.shape[1]

    def body(
        x_ref, wq_ref, wo_ref, k_ref, v_ref, out_ref,
        q_buf, acc, send, recv, norm_buf,
        send_sem, recv_sem,
    ):
        me = lax.axis_index("i")

        barrier = pltpu.get_barrier_semaphore()
        for k in range(N_STEPS):
            pl.semaphore_signal(
                barrier, inc=1,
                device_id=(me ^ (1 << k),),
                device_id_type=pl.DeviceIdType.MESH,
            )

        wq = wq_ref[...].astype(BF16)
        for b in range(B):
            q_buf[b] = lax.dot_general(
                x_ref[b].astype(BF16), wq,
                (((1,), (0,)), ((), ())),
                preferred_element_type=F32,
            ).astype(BF16)

        lane = lax.broadcasted_iota(jnp.int32, (skv, DE - Dh), 1)
        ext = (lane == 0).astype(BF16)

        def compute_partials(b):
            for g in range(Hkv):
                kk = k_ref[b, :, g, :].astype(BF16)
                vve = jnp.concatenate(
                    [v_ref[b, :, g, :].astype(BF16), ext], axis=1
                )
                for h in range(g * GROUP, (g + 1) * GROUP):
                    q = q_buf[b, :, h * Dh:(h + 1) * Dh]
                    s = lax.dot_general(
                        q, kk, (((1,), (1,)), ((), ())),
                        preferred_element_type=F32,
                    ) * SCALE
                    p = jnp.exp(s)
                    acc[b, :, h * DE:(h + 1) * DE] = lax.dot_general(
                        p.astype(BF16), vve, (((1,), (0,)), ((), ())),
                        preferred_element_type=F32,
                    )

        def make_rdma(c, k):
            return pltpu.make_async_remote_copy(
                src_ref=send.at[c], dst_ref=recv.at[c, k],
                send_sem=send_sem.at[c, k], recv_sem=recv_sem.at[c, k],
                device_id=(me ^ (1 << k),),
                device_id_type=pl.DeviceIdType.MESH,
            )

        def start(c, k):
            send[c] = acc[c].astype(BF16)
            d = make_rdma(c, k)
            d.start()
            return d

        wo = wo_ref[...].astype(BF16)

        def finalize(b):
            for h in range(Hq):
                o = acc[b, :, h * DE:h * DE + Dh]
                l = acc[b, :, h * DE + Dh:h * DE + Dh + 1]
                norm_buf[b, :, h * Dh:(h + 1) * Dh] = (o / l).astype(BF16)
            out_ref[b] = lax.dot_general(
                norm_buf[b], wo, (((1,), (0,)), ((), ())),
                preferred_element_type=F32,
            )

        d = [[None] * N_STEPS for _ in range(B)]
        compute_partials(0)
        pl.semaphore_wait(barrier, N_STEPS)
        d[0][0] = start(0, 0)
        compute_partials(1)
        d[1][0] = start(1, 0)
        for k in range(N_STEPS):
            for c in range(B):
                d[c][k].wait()
                acc[c] = acc[c] + recv[c, k].astype(F32)
                if k < N_STEPS - 1:
                    d[c][k + 1] = start(c, k + 1)
                else:
                    finalize(c)

    return pl.pallas_call(
        body,
        out_shape=jax.ShapeDtypeStruct((B, Sq, D_MODEL), F32),
        in_specs=[pl.BlockSpec(memory_space=pltpu.VMEM)] * 5,
        out_specs=pl.BlockSpec(memory_space=pltpu.VMEM),
        scratch_shapes=[
            pltpu.VMEM((B, Sq, DQ), BF16),
            pltpu.VMEM((B, Sq, DQE), F32),
            pltpu.VMEM((B, Sq, DQE), BF16),
            pltpu.VMEM((B, N_STEPS, Sq, DQE), BF16),
            pltpu.VMEM((B, Sq, DQ), BF16),
            pltpu.SemaphoreType.DMA((B, N_STEPS)),
            pltpu.SemaphoreType.DMA((B, N_STEPS)),
        ],
        compiler_params=pltpu.CompilerParams(collective_id=0),
    )(x, Wq, Wo, K_ext, V_ext)
